# baseline (speedup 1.0000x reference)
import functools

import jax
import jax.numpy as jnp
from jax import lax
from jax.experimental import pallas as pl
from jax.experimental.pallas import tpu as pltpu
from jax.experimental.pallas import tpu_sc as plsc

N = 10000
E = 320000
H = 128
C = 40

NC = 2
NS = 16
NW = NC * NS
E_PER_W = E // NW
CHUNK = 80
NCHUNK = E_PER_W // CHUNK
NP = 10112
ROWS_PER_S = NP // NS


def _ln(x, g, b):
    mu = jnp.mean(x, axis=-1, keepdims=True)
    var = jnp.mean((x - mu) ** 2, axis=-1, keepdims=True)
    return (x - mu) * lax.rsqrt(var + 1e-5) * g[None, :] + b[None, :]


def _make_sc_cnt():
    mesh = plsc.VectorSubcoreMesh(
        core_axis_name="c", subcore_axis_name="s", num_cores=NC, num_subcores=NS
    )

    @functools.partial(
        pl.kernel,
        out_type=(jax.ShapeDtypeStruct((NW, NP), jnp.float32),),
        mesh=mesh,
        scratch_types=(
            pltpu.VMEM((E_PER_W,), jnp.int32),
            pltpu.VMEM((NP,), jnp.float32),
        ),
        compiler_params=pltpu.CompilerParams(needs_layout_passes=False),
    )
    def sc_cnt(dst_hbm, zeros1_hbm, cnt_hbm, dstall, cvt):
        c = lax.axis_index("c")
        s = lax.axis_index("s")
        wid = c * NS + s
        base = wid * E_PER_W
        pltpu.sync_copy(zeros1_hbm, cvt)
        pltpu.sync_copy(dst_hbm.at[pl.ds(base, E_PER_W)], dstall)
        ones16 = jnp.full((16,), 1.0, jnp.float32)

        @pl.loop(0, E_PER_W // 16, unroll=4)
        def _vec(v):
            plsc.addupdate_scatter(cvt, [dstall[pl.ds(v * 16, 16)]], ones16)

        pltpu.sync_copy(cvt, cnt_hbm.at[wid])

    return sc_cnt


_sc_cnt = _make_sc_cnt()


def _make_sc_agg():
    mesh = plsc.VectorSubcoreMesh(
        core_axis_name="c", subcore_axis_name="s", num_cores=NC, num_subcores=NS
    )
    scratch = [
        pltpu.VMEM((CHUNK,), jnp.int32),
        pltpu.VMEM((CHUNK,), jnp.int32),
        pltpu.VMEM((CHUNK,), jnp.int32),
        pltpu.VMEM((CHUNK,), jnp.int32),
        pltpu.VMEM((CHUNK, H), jnp.float32),
        pltpu.VMEM((CHUNK, H), jnp.float32),
        pltpu.VMEM_SHARED((NP, H), jnp.float32),
        pltpu.SemaphoreType.DMA,
        pltpu.SemaphoreType.DMA,
        pltpu.SemaphoreType.DMA,
        pltpu.SemaphoreType.DMA,
        pltpu.SemaphoreType.DMA,
        pltpu.SemaphoreType.DMA,
        pltpu.SemaphoreType.DMA,
        pltpu.SemaphoreType.DMA,
    ]

    @functools.partial(
        pl.kernel,
        out_type=(jax.ShapeDtypeStruct((NC, NP, H), jnp.float32),),
        mesh=mesh,
        scratch_types=tuple(scratch),
        compiler_params=pltpu.CompilerParams(needs_layout_passes=False),
    )
    def sc_agg(h_hbm, src_hbm, dst_hbm, zeros_hbm, parts_hbm,
               si_a, si_b, di_a, di_b, rows_a, rows_b, acc,
               gsem_a, gsem_b, sisem_a, sisem_b, disem_a, disem_b,
               ssem_a, ssem_b):
        c = lax.axis_index("c")
        s = lax.axis_index("s")
        wid = c * NS + s
        row0 = s * ROWS_PER_S
        pltpu.sync_copy(zeros_hbm.at[pl.ds(row0, ROWS_PER_S)],
                        acc.at[pl.ds(row0, ROWS_PER_S)])
        base = wid * E_PER_W
        plsc.subcore_barrier()

        def src_copy(j, si, sem):
            return pltpu.make_async_copy(
                src_hbm.at[pl.ds(base + j * CHUNK, CHUNK)], si, sem)

        def dst_copy(j, di, sem):
            return pltpu.make_async_copy(
                dst_hbm.at[pl.ds(base + j * CHUNK, CHUNK)], di, sem)

        def gather_start(si, buf, sem):
            pltpu.async_copy(h_hbm.at[si], buf, sem)

        def gather_wait(si, buf, sem):
            pltpu.make_async_copy(h_hbm.at[si], buf, sem).wait()

        src_copy(0, si_a, sisem_a).start()
        dst_copy(0, di_a, disem_a).start()
        src_copy(1, si_b, sisem_b).start()
        dst_copy(1, di_b, disem_b).start()
        src_copy(0, si_a, sisem_a).wait()
        gather_start(si_a, rows_a, gsem_a)
        src_copy(1, si_b, sisem_b).wait()

        def scatter_start(buf, di, sem):
            pltpu.async_copy(buf, acc.at[di], sem, add=True)

        def scatter_wait(buf, di, sem):
            pltpu.make_async_copy(buf, acc.at[di], sem).wait()

        @pl.loop(0, NCHUNK // 2)
        def _pair(k):
            j0 = 2 * k

            @pl.when(j0 > 0)
            def _():
                src_copy(j0 + 1, si_b, sisem_b).wait()

            gather_start(si_b, rows_b, gsem_b)
            gather_wait(si_a, rows_a, gsem_a)

            @pl.when(j0 + 2 < NCHUNK)
            def _():
                src_copy(j0 + 2, si_a, sisem_a).start()

            dst_copy(j0, di_a, disem_a).wait()
            scatter_start(rows_a, di_a, ssem_a)

            gather_wait(si_b, rows_b, gsem_b)

            @pl.when(j0 + 3 < NCHUNK)
            def _():
                src_copy(j0 + 3, si_b, sisem_b).start()

            dst_copy(j0 + 1, di_b, disem_b).wait()
            scatter_start(rows_b, di_b, ssem_b)

            scatter_wait(rows_a, di_a, ssem_a)

            @pl.when(j0 + 2 < NCHUNK)
            def _():
                dst_copy(j0 + 2, di_a, disem_a).start()
                src_copy(j0 + 2, si_a, sisem_a).wait()
                gather_start(si_a, rows_a, gsem_a)

            scatter_wait(rows_b, di_b, ssem_b)

            @pl.when(j0 + 3 < NCHUNK)
            def _():
                dst_copy(j0 + 3, di_b, disem_b).start()

        if NCHUNK % 2 == 1:
            j = NCHUNK - 1
            gather_wait(si_a, rows_a, gsem_a)
            dst_copy(j, di_a, disem_a).wait()
            pltpu.sync_copy(rows_a, acc.at[di_a], add=True)

        plsc.subcore_barrier()
        pltpu.sync_copy(acc.at[pl.ds(row0, ROWS_PER_S)],
                        parts_hbm.at[c, pl.ds(row0, ROWS_PER_S)])

    return sc_agg


_sc_agg = _make_sc_agg()


_RB = 1024
_GRID = (N + _RB - 1) // _RB

_w_spec = pl.BlockSpec((H, H), lambda i: (0, 0))
_v_spec = pl.BlockSpec((H,), lambda i: (0,))
_h_spec = pl.BlockSpec((_RB, H), lambda i: (i, 0))


def _proj_body(x_ref, w1, b1, w2, b2, g, b, out_ref):
    xb = x_ref[...]
    h1 = jnp.maximum(
        jnp.dot(xb, w1[...], preferred_element_type=jnp.float32) + b1[...][None, :], 0.0)
    h2 = jnp.dot(h1, w2[...], preferred_element_type=jnp.float32) + b2[...][None, :]
    out_ref[...] = _ln(h2, g[...], b[...])


_tc_proj = pl.pallas_call(
    _proj_body,
    grid=(_GRID,),
    in_specs=[_h_spec, _w_spec, _v_spec, _w_spec, _v_spec, _v_spec, _v_spec],
    out_specs=_h_spec,
    out_shape=jax.ShapeDtypeStruct((N, H), jnp.float32),
)


def _update_body(h_ref, parts_ref, cnt_ref, wl, bl, wr, g, b, out_ref):
    hb = h_ref[...]
    agg = parts_ref[0] + parts_ref[1]
    cnt = jnp.sum(cnt_ref[...], axis=0)[:, None]
    agg = agg / jnp.maximum(cnt, 1.0)
    hn = (jnp.dot(agg, wl[...], preferred_element_type=jnp.float32)
          + bl[...][None, :]
          + jnp.dot(hb, wr[...], preferred_element_type=jnp.float32))
    hn = jnp.maximum(hn, 0.0)
    out_ref[...] = _ln(hb + hn, g[...], b[...])


_parts_spec = pl.BlockSpec((NC, _RB, H), lambda i: (0, i, 0))
_cnt_spec = pl.BlockSpec((NW, _RB), lambda i: (0, i))

_tc_update = pl.pallas_call(
    _update_body,
    grid=(_GRID,),
    in_specs=[_h_spec, _parts_spec, _cnt_spec, _w_spec, _v_spec, _w_spec,
              _v_spec, _v_spec],
    out_specs=_h_spec,
    out_shape=jax.ShapeDtypeStruct((N, H), jnp.float32),
)


def _update_cls_body(h_ref, parts_ref, cnt_ref, wl, bl, wr, g, b,
                     cw1, cb1, cw2, cb2, ze_ref, lg_ref):
    hb = h_ref[...]
    agg = parts_ref[0] + parts_ref[1]
    cnt = jnp.sum(cnt_ref[...], axis=0)[:, None]
    agg = agg / jnp.maximum(cnt, 1.0)
    hn = (jnp.dot(agg, wl[...], preferred_element_type=jnp.float32)
          + bl[...][None, :]
          + jnp.dot(hb, wr[...], preferred_element_type=jnp.float32))
    hn = jnp.maximum(hn, 0.0)
    ze = _ln(hb + hn, g[...], b[...])
    ze_ref[...] = ze
    t = jnp.maximum(
        jnp.dot(ze, cw1[...], preferred_element_type=jnp.float32) + cb1[...][None, :], 0.0)
    lg_ref[...] = jnp.dot(t, cw2[...], preferred_element_type=jnp.float32) + cb2[...][None, :]


_tc_update_cls = pl.pallas_call(
    _update_cls_body,
    grid=(_GRID,),
    in_specs=[_h_spec, _parts_spec, _cnt_spec, _w_spec, _v_spec, _w_spec,
              _v_spec, _v_spec,
              _w_spec, _v_spec, pl.BlockSpec((H, C), lambda i: (0, 0)),
              pl.BlockSpec((C,), lambda i: (0,))],
    out_specs=[_h_spec, pl.BlockSpec((_RB, C), lambda i: (i, 0))],
    out_shape=[jax.ShapeDtypeStruct((N, H), jnp.float32),
               jax.ShapeDtypeStruct((N, C), jnp.float32)],
)


def kernel(x, edge_index, ip_w1, ip_b1, ip_w2, ip_b2, in_g, in_b,
           sage_wl, sage_bl, sage_wr, ln_g, ln_b,
           cl_w1, cl_b1, cl_w2, cl_b2):
    ei = edge_index.astype(jnp.int32)
    src = ei[0]
    dst = ei[1]
    zeros = jnp.zeros((NP, H), jnp.float32)
    zeros1 = jnp.zeros((NP,), jnp.float32)

    (cnt,) = _sc_cnt(dst, zeros1)
    h = _tc_proj(x, ip_w1, ip_b1, ip_w2, ip_b2, in_g, in_b)
    (parts,) = _sc_agg(h, src, dst, zeros)
    h = _tc_update(h, parts, cnt, sage_wl[0], sage_bl[0], sage_wr[0],
                   ln_g[0], ln_b[0])
    (parts2,) = _sc_agg(h, src, dst, zeros)
    ze, logits = _tc_update_cls(h, parts2, cnt, sage_wl[1], sage_bl[1],
                                sage_wr[1], ln_g[1], ln_b[1],
                                cl_w1, cl_b1, cl_w2, cl_b2)
    return (logits, ze)


# speedup vs baseline: 1.2872x; 1.2872x over previous
import functools

import jax
import jax.numpy as jnp
from jax import lax
from jax.experimental import pallas as pl
from jax.experimental.pallas import tpu as pltpu
from jax.experimental.pallas import tpu_sc as plsc

N = 10000
E = 320000
H = 128
C = 40

NC = 2
NS = 16
NW = NC * NS
E_PER_W = E // NW
CHUNK = 80
NCHUNK = E_PER_W // CHUNK
CA = 128
NCH_G = E // CA
NLOC = NCH_G // NW
NTAIL = NCH_G - NLOC * NW
NP = 10112
ROWS_PER_S = NP // NS


def _ln(x, g, b):
    mu = jnp.mean(x, axis=-1, keepdims=True)
    var = jnp.mean((x - mu) ** 2, axis=-1, keepdims=True)
    return (x - mu) * lax.rsqrt(var + 1e-5) * g[None, :] + b[None, :]


def _make_sc_cnt():
    mesh = plsc.VectorSubcoreMesh(
        core_axis_name="c", subcore_axis_name="s", num_cores=NC, num_subcores=NS
    )

    @functools.partial(
        pl.kernel,
        out_type=(jax.ShapeDtypeStruct((NW, NP), jnp.float32),),
        mesh=mesh,
        scratch_types=(
            pltpu.VMEM((E_PER_W,), jnp.int32),
            pltpu.VMEM((NP,), jnp.float32),
        ),
        compiler_params=pltpu.CompilerParams(needs_layout_passes=False),
    )
    def sc_cnt(dst_hbm, zeros1_hbm, cnt_hbm, dstall, cvt):
        c = lax.axis_index("c")
        s = lax.axis_index("s")
        wid = c * NS + s
        base = wid * E_PER_W
        pltpu.sync_copy(zeros1_hbm, cvt)
        pltpu.sync_copy(dst_hbm.at[pl.ds(base, E_PER_W)], dstall)
        ones16 = jnp.full((16,), 1.0, jnp.float32)

        @pl.loop(0, E_PER_W // 16, unroll=4)
        def _vec(v):
            plsc.addupdate_scatter(cvt, [dstall[pl.ds(v * 16, 16)]], ones16)

        pltpu.sync_copy(cvt, cnt_hbm.at[wid])

    return sc_cnt


_sc_cnt = _make_sc_cnt()


def _make_sc_agg():
    mesh = plsc.VectorSubcoreMesh(
        core_axis_name="c", subcore_axis_name="s", num_cores=NC, num_subcores=NS
    )
    scratch = [
        pltpu.VMEM((CA,), jnp.int32),
        pltpu.VMEM((CA,), jnp.int32),
        pltpu.VMEM((CA,), jnp.int32),
        pltpu.VMEM((CA,), jnp.int32),
        pltpu.VMEM((CA, H), jnp.float32),
        pltpu.VMEM((CA, H), jnp.float32),
        pltpu.VMEM_SHARED((NP, H), jnp.float32),
        pltpu.SemaphoreType.DMA,
        pltpu.SemaphoreType.DMA,
        pltpu.SemaphoreType.DMA,
        pltpu.SemaphoreType.DMA,
        pltpu.SemaphoreType.DMA,
        pltpu.SemaphoreType.DMA,
    ]

    @functools.partial(
        pl.kernel,
        out_type=(jax.ShapeDtypeStruct((NC, NP, H), jnp.float32),),
        mesh=mesh,
        scratch_types=tuple(scratch),
        compiler_params=pltpu.CompilerParams(needs_layout_passes=False),
    )
    def sc_agg(h_hbm, src_hbm, dst_hbm, zeros_hbm, parts_hbm,
               si_a, si_b, di_a, di_b, rows_a, rows_b, acc,
               gsem_a, gsem_b, sisem_a, sisem_b, disem_a, disem_b):
        c = lax.axis_index("c")
        s = lax.axis_index("s")
        wid = c * NS + s
        row0 = s * ROWS_PER_S
        pltpu.sync_copy(zeros_hbm.at[pl.ds(row0, ROWS_PER_S)],
                        acc.at[pl.ds(row0, ROWS_PER_S)])
        plsc.subcore_barrier()

        def src_copy(j, si, sem):
            return pltpu.make_async_copy(
                src_hbm.at[pl.ds((wid + j * NW) * CA, CA)], si, sem)

        def dst_copy(j, di, sem):
            return pltpu.make_async_copy(
                dst_hbm.at[pl.ds((wid + j * NW) * CA, CA)], di, sem)

        def gather_start(si, buf, sem):
            pltpu.async_copy(h_hbm.at[si], buf, sem)

        def gather_wait(si, buf, sem):
            pltpu.make_async_copy(h_hbm.at[si], buf, sem).wait()

        src_copy(0, si_a, sisem_a).start()
        dst_copy(0, di_a, disem_a).start()
        src_copy(1, si_b, sisem_b).start()
        dst_copy(1, di_b, disem_b).start()
        src_copy(0, si_a, sisem_a).wait()
        gather_start(si_a, rows_a, gsem_a)
        src_copy(1, si_b, sisem_b).wait()

        @pl.loop(0, NLOC // 2)
        def _pair(k):
            j0 = 2 * k

            @pl.when(j0 > 0)
            def _():
                src_copy(j0 + 1, si_b, sisem_b).wait()

            gather_start(si_b, rows_b, gsem_b)
            gather_wait(si_a, rows_a, gsem_a)

            @pl.when(j0 + 2 < NLOC)
            def _():
                src_copy(j0 + 2, si_a, sisem_a).start()

            dst_copy(j0, di_a, disem_a).wait()
            pltpu.sync_copy(rows_a, acc.at[di_a], add=True)

            @pl.when(j0 + 2 < NLOC)
            def _():
                dst_copy(j0 + 2, di_a, disem_a).start()
                src_copy(j0 + 2, si_a, sisem_a).wait()
                gather_start(si_a, rows_a, gsem_a)

            gather_wait(si_b, rows_b, gsem_b)

            @pl.when(j0 + 3 < NLOC)
            def _():
                src_copy(j0 + 3, si_b, sisem_b).start()

            dst_copy(j0 + 1, di_b, disem_b).wait()
            pltpu.sync_copy(rows_b, acc.at[di_b], add=True)

            @pl.when(j0 + 3 < NLOC)
            def _():
                dst_copy(j0 + 3, di_b, disem_b).start()

        @pl.when(wid < NTAIL)
        def _tail():
            g0 = NLOC * NW + wid
            pltpu.sync_copy(src_hbm.at[pl.ds(g0 * CA, CA)], si_a)
            pltpu.sync_copy(dst_hbm.at[pl.ds(g0 * CA, CA)], di_a)
            pltpu.async_copy(h_hbm.at[si_a], rows_a, gsem_a).wait()
            pltpu.sync_copy(rows_a, acc.at[di_a], add=True)

        plsc.subcore_barrier()
        pltpu.sync_copy(acc.at[pl.ds(row0, ROWS_PER_S)],
                        parts_hbm.at[c, pl.ds(row0, ROWS_PER_S)])

    return sc_agg


_sc_agg = _make_sc_agg()


_RB = 1024
_GRID = (N + _RB - 1) // _RB

_w_spec = pl.BlockSpec((H, H), lambda i: (0, 0))
_v_spec = pl.BlockSpec((H,), lambda i: (0,))
_h_spec = pl.BlockSpec((_RB, H), lambda i: (i, 0))


def _proj_body(x_ref, w1, b1, w2, b2, g, b, out_ref):
    xb = x_ref[...]
    h1 = jnp.maximum(
        jnp.dot(xb, w1[...], preferred_element_type=jnp.float32) + b1[...][None, :], 0.0)
    h2 = jnp.dot(h1, w2[...], preferred_element_type=jnp.float32) + b2[...][None, :]
    out_ref[...] = _ln(h2, g[...], b[...])


_tc_proj = pl.pallas_call(
    _proj_body,
    grid=(_GRID,),
    in_specs=[_h_spec, _w_spec, _v_spec, _w_spec, _v_spec, _v_spec, _v_spec],
    out_specs=_h_spec,
    out_shape=jax.ShapeDtypeStruct((N, H), jnp.float32),
)


def _update_body(h_ref, parts_ref, cnt_ref, wl, bl, wr, g, b, out_ref):
    hb = h_ref[...]
    agg = parts_ref[0] + parts_ref[1]
    cnt = jnp.sum(cnt_ref[...], axis=0)[:, None]
    agg = agg / jnp.maximum(cnt, 1.0)
    hn = (jnp.dot(agg, wl[...], preferred_element_type=jnp.float32)
          + bl[...][None, :]
          + jnp.dot(hb, wr[...], preferred_element_type=jnp.float32))
    hn = jnp.maximum(hn, 0.0)
    out_ref[...] = _ln(hb + hn, g[...], b[...])


_parts_spec = pl.BlockSpec((NC, _RB, H), lambda i: (0, i, 0))
_cnt_spec = pl.BlockSpec((NW, _RB), lambda i: (0, i))

_tc_update = pl.pallas_call(
    _update_body,
    grid=(_GRID,),
    in_specs=[_h_spec, _parts_spec, _cnt_spec, _w_spec, _v_spec, _w_spec,
              _v_spec, _v_spec],
    out_specs=_h_spec,
    out_shape=jax.ShapeDtypeStruct((N, H), jnp.float32),
)


def _update_cls_body(h_ref, parts_ref, cnt_ref, wl, bl, wr, g, b,
                     cw1, cb1, cw2, cb2, ze_ref, lg_ref):
    hb = h_ref[...]
    agg = parts_ref[0] + parts_ref[1]
    cnt = jnp.sum(cnt_ref[...], axis=0)[:, None]
    agg = agg / jnp.maximum(cnt, 1.0)
    hn = (jnp.dot(agg, wl[...], preferred_element_type=jnp.float32)
          + bl[...][None, :]
          + jnp.dot(hb, wr[...], preferred_element_type=jnp.float32))
    hn = jnp.maximum(hn, 0.0)
    ze = _ln(hb + hn, g[...], b[...])
    ze_ref[...] = ze
    t = jnp.maximum(
        jnp.dot(ze, cw1[...], preferred_element_type=jnp.float32) + cb1[...][None, :], 0.0)
    lg_ref[...] = jnp.dot(t, cw2[...], preferred_element_type=jnp.float32) + cb2[...][None, :]


_tc_update_cls = pl.pallas_call(
    _update_cls_body,
    grid=(_GRID,),
    in_specs=[_h_spec, _parts_spec, _cnt_spec, _w_spec, _v_spec, _w_spec,
              _v_spec, _v_spec,
              _w_spec, _v_spec, pl.BlockSpec((H, C), lambda i: (0, 0)),
              pl.BlockSpec((C,), lambda i: (0,))],
    out_specs=[_h_spec, pl.BlockSpec((_RB, C), lambda i: (i, 0))],
    out_shape=[jax.ShapeDtypeStruct((N, H), jnp.float32),
               jax.ShapeDtypeStruct((N, C), jnp.float32)],
)


def kernel(x, edge_index, ip_w1, ip_b1, ip_w2, ip_b2, in_g, in_b,
           sage_wl, sage_bl, sage_wr, ln_g, ln_b,
           cl_w1, cl_b1, cl_w2, cl_b2):
    ei = edge_index.astype(jnp.int32)
    src = ei[0]
    dst = ei[1]
    zeros = jnp.zeros((NP, H), jnp.float32)
    zeros1 = jnp.zeros((NP,), jnp.float32)

    (cnt,) = _sc_cnt(dst, zeros1)
    h = _tc_proj(x, ip_w1, ip_b1, ip_w2, ip_b2, in_g, in_b)
    (parts,) = _sc_agg(h, src, dst, zeros)
    h = _tc_update(h, parts, cnt, sage_wl[0], sage_bl[0], sage_wr[0],
                   ln_g[0], ln_b[0])
    (parts2,) = _sc_agg(h, src, dst, zeros)
    ze, logits = _tc_update_cls(h, parts2, cnt, sage_wl[1], sage_bl[1],
                                sage_wr[1], ln_g[1], ln_b[1],
                                cl_w1, cl_b1, cl_w2, cl_b2)
    return (logits, ze)


# speedup vs baseline: 1.3267x; 1.0307x over previous
import functools

import jax
import jax.numpy as jnp
from jax import lax
from jax.experimental import pallas as pl
from jax.experimental.pallas import tpu as pltpu
from jax.experimental.pallas import tpu_sc as plsc

N = 10000
E = 320000
H = 128
C = 40

NC = 2
NS = 16
NW = NC * NS
E_PER_W = E // NW
CHUNK = 80
NCHUNK = E_PER_W // CHUNK
CA = 128
NCH_G = E // CA
NLOC = NCH_G // NW
NTAIL = NCH_G - NLOC * NW
NP = 10112
ROWS_PER_S = NP // NS


def _ln(x, g, b):
    mu = jnp.mean(x, axis=-1, keepdims=True)
    var = jnp.mean((x - mu) ** 2, axis=-1, keepdims=True)
    return (x - mu) * lax.rsqrt(var + 1e-5) * g[None, :] + b[None, :]


def _make_sc_agg(with_cnt: bool):
    mesh = plsc.VectorSubcoreMesh(
        core_axis_name="c", subcore_axis_name="s", num_cores=NC, num_subcores=NS
    )
    out_type = [jax.ShapeDtypeStruct((NC, NP, H), jnp.float32)]
    if with_cnt:
        out_type.append(jax.ShapeDtypeStruct((NW, NP), jnp.float32))
    scratch = [
        pltpu.VMEM((CA,), jnp.int32),
        pltpu.VMEM((CA,), jnp.int32),
        pltpu.VMEM((CA,), jnp.int32),
        pltpu.VMEM((CA,), jnp.int32),
        pltpu.VMEM((CA, H), jnp.float32),
        pltpu.VMEM((CA, H), jnp.float32),
        pltpu.VMEM_SHARED((NP, H), jnp.float32),
        pltpu.SemaphoreType.DMA,
        pltpu.SemaphoreType.DMA,
        pltpu.SemaphoreType.DMA,
        pltpu.SemaphoreType.DMA,
        pltpu.SemaphoreType.DMA,
        pltpu.SemaphoreType.DMA,
    ]
    if with_cnt:
        scratch.append(pltpu.VMEM((NP,), jnp.float32))

    @functools.partial(
        pl.kernel,
        out_type=tuple(out_type),
        mesh=mesh,
        scratch_types=tuple(scratch),
        compiler_params=pltpu.CompilerParams(needs_layout_passes=False),
    )
    def sc_agg(*refs):
        if with_cnt:
            (h_hbm, src_hbm, dst_hbm, zeros_hbm, zeros1_hbm,
             parts_hbm, cnt_hbm,
             si_a, si_b, di_a, di_b, rows_a, rows_b, acc,
             gsem_a, gsem_b, sisem_a, sisem_b, disem_a, disem_b, cvt) = refs
        else:
            (h_hbm, src_hbm, dst_hbm, zeros_hbm, parts_hbm,
             si_a, si_b, di_a, di_b, rows_a, rows_b, acc,
             gsem_a, gsem_b, sisem_a, sisem_b, disem_a, disem_b) = refs
        c = lax.axis_index("c")
        s = lax.axis_index("s")
        wid = c * NS + s
        row0 = s * ROWS_PER_S
        pltpu.sync_copy(zeros_hbm.at[pl.ds(row0, ROWS_PER_S)],
                        acc.at[pl.ds(row0, ROWS_PER_S)])
        if with_cnt:
            pltpu.sync_copy(zeros1_hbm, cvt)
        plsc.subcore_barrier()
        ones16 = jnp.full((16,), 1.0, jnp.float32)

        def count(di):
            if with_cnt:
                for k in range(CA // 16):
                    plsc.addupdate_scatter(cvt, [di[pl.ds(k * 16, 16)]], ones16)

        def src_copy(j, si, sem):
            return pltpu.make_async_copy(
                src_hbm.at[pl.ds((wid + j * NW) * CA, CA)], si, sem)

        def dst_copy(j, di, sem):
            return pltpu.make_async_copy(
                dst_hbm.at[pl.ds((wid + j * NW) * CA, CA)], di, sem)

        def gather_start(si, buf, sem):
            pltpu.async_copy(h_hbm.at[si], buf, sem)

        def gather_wait(si, buf, sem):
            pltpu.make_async_copy(h_hbm.at[si], buf, sem).wait()

        src_copy(0, si_a, sisem_a).start()
        dst_copy(0, di_a, disem_a).start()
        src_copy(1, si_b, sisem_b).start()
        dst_copy(1, di_b, disem_b).start()
        src_copy(0, si_a, sisem_a).wait()
        gather_start(si_a, rows_a, gsem_a)
        src_copy(1, si_b, sisem_b).wait()

        @pl.loop(0, NLOC // 2)
        def _pair(k):
            j0 = 2 * k

            @pl.when(j0 > 0)
            def _():
                src_copy(j0 + 1, si_b, sisem_b).wait()

            gather_start(si_b, rows_b, gsem_b)
            gather_wait(si_a, rows_a, gsem_a)

            @pl.when(j0 + 2 < NLOC)
            def _():
                src_copy(j0 + 2, si_a, sisem_a).start()

            dst_copy(j0, di_a, disem_a).wait()
            count(di_a)
            pltpu.sync_copy(rows_a, acc.at[di_a], add=True)

            @pl.when(j0 + 2 < NLOC)
            def _():
                dst_copy(j0 + 2, di_a, disem_a).start()
                src_copy(j0 + 2, si_a, sisem_a).wait()
                gather_start(si_a, rows_a, gsem_a)

            gather_wait(si_b, rows_b, gsem_b)

            @pl.when(j0 + 3 < NLOC)
            def _():
                src_copy(j0 + 3, si_b, sisem_b).start()

            dst_copy(j0 + 1, di_b, disem_b).wait()
            count(di_b)
            pltpu.sync_copy(rows_b, acc.at[di_b], add=True)

            @pl.when(j0 + 3 < NLOC)
            def _():
                dst_copy(j0 + 3, di_b, disem_b).start()

        @pl.when(wid < NTAIL)
        def _tail():
            g0 = NLOC * NW + wid
            pltpu.sync_copy(src_hbm.at[pl.ds(g0 * CA, CA)], si_a)
            pltpu.sync_copy(dst_hbm.at[pl.ds(g0 * CA, CA)], di_a)
            count(di_a)
            pltpu.async_copy(h_hbm.at[si_a], rows_a, gsem_a).wait()
            pltpu.sync_copy(rows_a, acc.at[di_a], add=True)

        plsc.subcore_barrier()
        pltpu.sync_copy(acc.at[pl.ds(row0, ROWS_PER_S)],
                        parts_hbm.at[c, pl.ds(row0, ROWS_PER_S)])
        if with_cnt:
            pltpu.sync_copy(cvt, cnt_hbm.at[wid])

    return sc_agg


_sc_agg_cnt = _make_sc_agg(True)
_sc_agg = _make_sc_agg(False)


_RB = 1024
_GRID = (N + _RB - 1) // _RB

_w_spec = pl.BlockSpec((H, H), lambda i: (0, 0))
_v_spec = pl.BlockSpec((H,), lambda i: (0,))
_h_spec = pl.BlockSpec((_RB, H), lambda i: (i, 0))


def _proj_body(x_ref, w1, b1, w2, b2, g, b, out_ref):
    xb = x_ref[...]
    h1 = jnp.maximum(
        jnp.dot(xb, w1[...], preferred_element_type=jnp.float32) + b1[...][None, :], 0.0)
    h2 = jnp.dot(h1, w2[...], preferred_element_type=jnp.float32) + b2[...][None, :]
    out_ref[...] = _ln(h2, g[...], b[...])


_tc_proj = pl.pallas_call(
    _proj_body,
    grid=(_GRID,),
    in_specs=[_h_spec, _w_spec, _v_spec, _w_spec, _v_spec, _v_spec, _v_spec],
    out_specs=_h_spec,
    out_shape=jax.ShapeDtypeStruct((N, H), jnp.float32),
)


def _update_body(h_ref, parts_ref, cnt_ref, wl, bl, wr, g, b, out_ref):
    hb = h_ref[...]
    agg = parts_ref[0] + parts_ref[1]
    cnt = jnp.sum(cnt_ref[...], axis=0)[:, None]
    agg = agg / jnp.maximum(cnt, 1.0)
    hn = (jnp.dot(agg, wl[...], preferred_element_type=jnp.float32)
          + bl[...][None, :]
          + jnp.dot(hb, wr[...], preferred_element_type=jnp.float32))
    hn = jnp.maximum(hn, 0.0)
    out_ref[...] = _ln(hb + hn, g[...], b[...])


_parts_spec = pl.BlockSpec((NC, _RB, H), lambda i: (0, i, 0))
_cnt_spec = pl.BlockSpec((NW, _RB), lambda i: (0, i))

_tc_update = pl.pallas_call(
    _update_body,
    grid=(_GRID,),
    in_specs=[_h_spec, _parts_spec, _cnt_spec, _w_spec, _v_spec, _w_spec,
              _v_spec, _v_spec],
    out_specs=_h_spec,
    out_shape=jax.ShapeDtypeStruct((N, H), jnp.float32),
)


def _update_cls_body(h_ref, parts_ref, cnt_ref, wl, bl, wr, g, b,
                     cw1, cb1, cw2, cb2, ze_ref, lg_ref):
    hb = h_ref[...]
    agg = parts_ref[0] + parts_ref[1]
    cnt = jnp.sum(cnt_ref[...], axis=0)[:, None]
    agg = agg / jnp.maximum(cnt, 1.0)
    hn = (jnp.dot(agg, wl[...], preferred_element_type=jnp.float32)
          + bl[...][None, :]
          + jnp.dot(hb, wr[...], preferred_element_type=jnp.float32))
    hn = jnp.maximum(hn, 0.0)
    ze = _ln(hb + hn, g[...], b[...])
    ze_ref[...] = ze
    t = jnp.maximum(
        jnp.dot(ze, cw1[...], preferred_element_type=jnp.float32) + cb1[...][None, :], 0.0)
    lg_ref[...] = jnp.dot(t, cw2[...], preferred_element_type=jnp.float32) + cb2[...][None, :]


_tc_update_cls = pl.pallas_call(
    _update_cls_body,
    grid=(_GRID,),
    in_specs=[_h_spec, _parts_spec, _cnt_spec, _w_spec, _v_spec, _w_spec,
              _v_spec, _v_spec,
              _w_spec, _v_spec, pl.BlockSpec((H, C), lambda i: (0, 0)),
              pl.BlockSpec((C,), lambda i: (0,))],
    out_specs=[_h_spec, pl.BlockSpec((_RB, C), lambda i: (i, 0))],
    out_shape=[jax.ShapeDtypeStruct((N, H), jnp.float32),
               jax.ShapeDtypeStruct((N, C), jnp.float32)],
)


def kernel(x, edge_index, ip_w1, ip_b1, ip_w2, ip_b2, in_g, in_b,
           sage_wl, sage_bl, sage_wr, ln_g, ln_b,
           cl_w1, cl_b1, cl_w2, cl_b2):
    ei = edge_index.astype(jnp.int32)
    src = ei[0]
    dst = ei[1]
    zeros = jnp.zeros((NP, H), jnp.float32)
    zeros1 = jnp.zeros((NP,), jnp.float32)

    h = _tc_proj(x, ip_w1, ip_b1, ip_w2, ip_b2, in_g, in_b)
    parts, cnt = _sc_agg_cnt(h, src, dst, zeros, zeros1)
    h = _tc_update(h, parts, cnt, sage_wl[0], sage_bl[0], sage_wr[0],
                   ln_g[0], ln_b[0])
    (parts2,) = _sc_agg(h, src, dst, zeros)
    ze, logits = _tc_update_cls(h, parts2, cnt, sage_wl[1], sage_bl[1],
                                sage_wr[1], ln_g[1], ln_b[1],
                                cl_w1, cl_b1, cl_w2, cl_b2)
    return (logits, ze)


# speedup vs baseline: 1.3305x; 1.0028x over previous
import functools

import jax
import jax.numpy as jnp
from jax import lax
from jax.experimental import pallas as pl
from jax.experimental.pallas import tpu as pltpu
from jax.experimental.pallas import tpu_sc as plsc

N = 10000
E = 320000
H = 128
C = 40

NC = 2
NS = 16
NW = NC * NS
E_PER_W = E // NW
CA = 128
NCH_G = E // CA
NLOC = NCH_G // NW
NTAIL = NCH_G - NLOC * NW
NP = 10112
ROWS_PER_S = NP // NS


def _ln(x, g, b):
    mu = jnp.mean(x, axis=-1, keepdims=True)
    var = jnp.mean((x - mu) ** 2, axis=-1, keepdims=True)
    return (x - mu) * lax.rsqrt(var + 1e-5) * g[None, :] + b[None, :]


def _make_sc_agg(with_cnt: bool):
    mesh = plsc.VectorSubcoreMesh(
        core_axis_name="c", subcore_axis_name="s", num_cores=NC, num_subcores=NS
    )
    out_type = [jax.ShapeDtypeStruct((NC, NP, H), jnp.float32)]
    if with_cnt:
        out_type.append(jax.ShapeDtypeStruct((NW, NP), jnp.float32))
    scratch = [
        pltpu.VMEM((CA,), jnp.int32),
        pltpu.VMEM((CA,), jnp.int32),
        pltpu.VMEM((CA,), jnp.int32),
        pltpu.VMEM((CA,), jnp.int32),
        pltpu.VMEM((CA, H), jnp.float32),
        pltpu.VMEM((CA, H), jnp.float32),
        pltpu.VMEM_SHARED((NP, H), jnp.float32),
        pltpu.SemaphoreType.DMA,
        pltpu.SemaphoreType.DMA,
        pltpu.SemaphoreType.DMA,
        pltpu.SemaphoreType.DMA,
        pltpu.SemaphoreType.DMA,
        pltpu.SemaphoreType.DMA,
    ]
    if with_cnt:
        scratch.append(pltpu.VMEM((NP,), jnp.float32))

    @functools.partial(
        pl.kernel,
        out_type=tuple(out_type),
        mesh=mesh,
        scratch_types=tuple(scratch),
        compiler_params=pltpu.CompilerParams(needs_layout_passes=False),
    )
    def sc_agg(*refs):
        if with_cnt:
            (h_hbm, src_hbm, dst_hbm, zeros_hbm, zeros1_hbm,
             parts_hbm, cnt_hbm,
             si_a, si_b, di_a, di_b, rows_a, rows_b, acc,
             gsem_a, gsem_b, sisem_a, sisem_b, disem_a, disem_b, cvt) = refs
        else:
            (h_hbm, src_hbm, dst_hbm, zeros_hbm, parts_hbm,
             si_a, si_b, di_a, di_b, rows_a, rows_b, acc,
             gsem_a, gsem_b, sisem_a, sisem_b, disem_a, disem_b) = refs
        c = lax.axis_index("c")
        s = lax.axis_index("s")
        wid = c * NS + s
        row0 = s * ROWS_PER_S
        pltpu.sync_copy(zeros_hbm.at[pl.ds(row0, ROWS_PER_S)],
                        acc.at[pl.ds(row0, ROWS_PER_S)])
        if with_cnt:
            pltpu.sync_copy(zeros1_hbm, cvt)
        plsc.subcore_barrier()
        ones16 = jnp.full((16,), 1.0, jnp.float32)

        def count(di):
            if with_cnt:
                for k in range(CA // 16):
                    plsc.addupdate_scatter(cvt, [di[pl.ds(k * 16, 16)]], ones16)

        def src_copy(j, si, sem):
            return pltpu.make_async_copy(
                src_hbm.at[pl.ds((wid + j * NW) * CA, CA)], si, sem)

        def dst_copy(j, di, sem):
            return pltpu.make_async_copy(
                dst_hbm.at[pl.ds((wid + j * NW) * CA, CA)], di, sem)

        def gather_start(si, buf, sem):
            pltpu.async_copy(h_hbm.at[si], buf, sem)

        def gather_wait(si, buf, sem):
            pltpu.make_async_copy(h_hbm.at[si], buf, sem).wait()

        src_copy(0, si_a, sisem_a).start()
        dst_copy(0, di_a, disem_a).start()
        src_copy(1, si_b, sisem_b).start()
        dst_copy(1, di_b, disem_b).start()
        src_copy(0, si_a, sisem_a).wait()
        gather_start(si_a, rows_a, gsem_a)
        src_copy(1, si_b, sisem_b).wait()

        @pl.loop(0, NLOC // 2)
        def _pair(k):
            j0 = 2 * k

            @pl.when(j0 > 0)
            def _():
                src_copy(j0 + 1, si_b, sisem_b).wait()

            gather_start(si_b, rows_b, gsem_b)
            gather_wait(si_a, rows_a, gsem_a)

            @pl.when(j0 + 2 < NLOC)
            def _():
                src_copy(j0 + 2, si_a, sisem_a).start()

            dst_copy(j0, di_a, disem_a).wait()
            count(di_a)
            pltpu.sync_copy(rows_a, acc.at[di_a], add=True)

            @pl.when(j0 + 2 < NLOC)
            def _():
                dst_copy(j0 + 2, di_a, disem_a).start()
                src_copy(j0 + 2, si_a, sisem_a).wait()
                gather_start(si_a, rows_a, gsem_a)

            gather_wait(si_b, rows_b, gsem_b)

            @pl.when(j0 + 3 < NLOC)
            def _():
                src_copy(j0 + 3, si_b, sisem_b).start()

            dst_copy(j0 + 1, di_b, disem_b).wait()
            count(di_b)
            pltpu.sync_copy(rows_b, acc.at[di_b], add=True)

            @pl.when(j0 + 3 < NLOC)
            def _():
                dst_copy(j0 + 3, di_b, disem_b).start()

        @pl.when(wid < NTAIL)
        def _tail():
            g0 = NLOC * NW + wid
            pltpu.sync_copy(src_hbm.at[pl.ds(g0 * CA, CA)], si_a)
            pltpu.sync_copy(dst_hbm.at[pl.ds(g0 * CA, CA)], di_a)
            count(di_a)
            pltpu.async_copy(h_hbm.at[si_a], rows_a, gsem_a).wait()
            pltpu.sync_copy(rows_a, acc.at[di_a], add=True)

        plsc.subcore_barrier()
        pltpu.sync_copy(acc.at[pl.ds(row0, ROWS_PER_S)],
                        parts_hbm.at[c, pl.ds(row0, ROWS_PER_S)])
        if with_cnt:
            pltpu.sync_copy(cvt, cnt_hbm.at[wid])

    return sc_agg


_sc_agg_cnt = _make_sc_agg(True)
_sc_agg = _make_sc_agg(False)


_RB = 1024
_GRID = (N + _RB - 1) // _RB

_w_spec = pl.BlockSpec((H, H), lambda i: (0, 0))
_v_spec = pl.BlockSpec((H,), lambda i: (0,))
_h_spec = pl.BlockSpec((_RB, H), lambda i: (i, 0))


def _proj_body(x_ref, w1, b1, w2, b2, g, b, out_ref):
    xb = x_ref[...]
    h1 = jnp.maximum(
        jnp.dot(xb, w1[...], preferred_element_type=jnp.float32) + b1[...][None, :], 0.0)
    h2 = jnp.dot(h1, w2[...], preferred_element_type=jnp.float32) + b2[...][None, :]
    out_ref[...] = _ln(h2, g[...], b[...])


_tc_proj = pl.pallas_call(
    _proj_body,
    grid=(_GRID,),
    in_specs=[_h_spec, _w_spec, _v_spec, _w_spec, _v_spec, _v_spec, _v_spec],
    out_specs=_h_spec,
    out_shape=jax.ShapeDtypeStruct((N, H), jnp.float32),
)


def _update_body(h_ref, parts_ref, cnt_ref, wl, bl, wr, g, b, out_ref):
    hb = h_ref[...]
    agg = parts_ref[0] + parts_ref[1]
    cnt = jnp.sum(cnt_ref[...], axis=0)[:, None]
    agg = agg / jnp.maximum(cnt, 1.0)
    hn = (jnp.dot(agg, wl[...], preferred_element_type=jnp.float32)
          + bl[...][None, :]
          + jnp.dot(hb, wr[...], preferred_element_type=jnp.float32))
    hn = jnp.maximum(hn, 0.0)
    out_ref[...] = _ln(hb + hn, g[...], b[...])


_parts_spec = pl.BlockSpec((NC, _RB, H), lambda i: (0, i, 0))
_cnt_spec = pl.BlockSpec((NW, _RB), lambda i: (0, i))

_tc_update = pl.pallas_call(
    _update_body,
    grid=(_GRID,),
    in_specs=[_h_spec, _parts_spec, _cnt_spec, _w_spec, _v_spec, _w_spec,
              _v_spec, _v_spec],
    out_specs=_h_spec,
    out_shape=jax.ShapeDtypeStruct((N, H), jnp.float32),
)


def _update_cls_body(h_ref, parts_ref, cnt_ref, wl, bl, wr, g, b,
                     cw1, cb1, cw2, cb2, ze_ref, lg_ref):
    hb = h_ref[...]
    agg = parts_ref[0] + parts_ref[1]
    cnt = jnp.sum(cnt_ref[...], axis=0)[:, None]
    agg = agg / jnp.maximum(cnt, 1.0)
    hn = (jnp.dot(agg, wl[...], preferred_element_type=jnp.float32)
          + bl[...][None, :]
          + jnp.dot(hb, wr[...], preferred_element_type=jnp.float32))
    hn = jnp.maximum(hn, 0.0)
    ze = _ln(hb + hn, g[...], b[...])
    ze_ref[...] = ze
    t = jnp.maximum(
        jnp.dot(ze, cw1[...], preferred_element_type=jnp.float32) + cb1[...][None, :], 0.0)
    lg_ref[...] = jnp.dot(t, cw2[...], preferred_element_type=jnp.float32) + cb2[...][None, :]


_tc_update_cls = pl.pallas_call(
    _update_cls_body,
    grid=(_GRID,),
    in_specs=[_h_spec, _parts_spec, _cnt_spec, _w_spec, _v_spec, _w_spec,
              _v_spec, _v_spec,
              _w_spec, _v_spec, pl.BlockSpec((H, C), lambda i: (0, 0)),
              pl.BlockSpec((C,), lambda i: (0,))],
    out_specs=[_h_spec, pl.BlockSpec((_RB, C), lambda i: (i, 0))],
    out_shape=[jax.ShapeDtypeStruct((N, H), jnp.float32),
               jax.ShapeDtypeStruct((N, C), jnp.float32)],
)


def kernel(x, edge_index, ip_w1, ip_b1, ip_w2, ip_b2, in_g, in_b,
           sage_wl, sage_bl, sage_wr, ln_g, ln_b,
           cl_w1, cl_b1, cl_w2, cl_b2):
    ei = edge_index.astype(jnp.int32)
    src = ei[0]
    dst = ei[1]
    zeros = jnp.zeros((NP, H), jnp.float32)
    zeros1 = jnp.zeros((NP,), jnp.float32)

    h = _tc_proj(x, ip_w1, ip_b1, ip_w2, ip_b2, in_g, in_b)
    parts, cnt = _sc_agg_cnt(h, src, dst, zeros, zeros1)
    h = _tc_update(h, parts, cnt, sage_wl[0], sage_bl[0], sage_wr[0],
                   ln_g[0], ln_b[0])
    (parts2,) = _sc_agg(h, src, dst, zeros)
    ze, logits = _tc_update_cls(h, parts2, cnt, sage_wl[1], sage_bl[1],
                                sage_wr[1], ln_g[1], ln_b[1],
                                cl_w1, cl_b1, cl_w2, cl_b2)
    return (logits, ze)
